# Initial kernel scaffold; baseline (speedup 1.0000x reference)
import jax
import jax.numpy as jnp
from jax.experimental import pallas as pl


def kernel(fluidFeatures, fluid_edge_index, fluid_edge_distances, W0, W1, W2, W3):
    raise NotImplementedError("write your pallas kernel here")


# speedup vs baseline: 2.3777x; 2.3777x over previous
import functools

import jax
import jax.numpy as jnp
from jax import lax
from jax.experimental import pallas as pl
from jax.experimental.pallas import tpu as pltpu
from jax.experimental.pallas import tpu_sc as plsc

N = 10000
E = 320000
SCALE = 1.0 / 128.0

NC = 2
NS = 16
NW = NC * NS
C = 64
NCH = E // C
RT = 624
ZR = 208
NB = 2


def _prep_body(col_ref, dx_ref, dy_ref, idx_ref, w_ref):
    col = col_ref[...]
    dx = dx_ref[...]
    dy = dy_ref[...]
    tx = (dx + 1.0) * 1.5
    ty = (dy + 1.0) * 1.5
    cxf = jnp.clip(jnp.floor(tx), 0.0, 2.0)
    cyf = jnp.clip(jnp.floor(ty), 0.0, 2.0)
    fx = tx - cxf
    fy = ty - cyf
    cx = cxf.astype(jnp.int32)
    cy = cyf.astype(jnp.int32)
    idx_ref[0] = col * 9 + cx * 3 + cy
    wx0 = 1.0 - fx
    wy0 = 1.0 - fy
    w_ref[0] = wx0 * wy0
    w_ref[1] = wx0 * fy
    w_ref[2] = fx * wy0
    w_ref[3] = fx * fy


_ER, _EC = 2500, 128


def _edge_prep(col_r, dx_r, dy_r):
    return pl.pallas_call(
        _prep_body,
        out_shape=(
            jax.ShapeDtypeStruct((1, _ER, _EC), jnp.int32),
            jax.ShapeDtypeStruct((4, _ER, _EC), jnp.float32),
        ),
    )(col_r, dx_r, dy_r)


def _mm_body(x_ref, w_ref, o_ref):
    o_ref[...] = jnp.dot(x_ref[...], w_ref[...],
                         preferred_element_type=jnp.float32)


def _mm_relu_body(a_ref, b_ref, w_ref, o_ref):
    x = jnp.maximum(a_ref[...] + b_ref[...], 0.0)
    o_ref[...] = jnp.dot(x, w_ref[...], preferred_element_type=jnp.float32)


_RB = 1000


def _mm0(x, wf):
    din, dcols = wf.shape
    return pl.pallas_call(
        _mm_body,
        grid=(N // _RB,),
        in_specs=[
            pl.BlockSpec((_RB, din), lambda i: (i, 0)),
            pl.BlockSpec((din, dcols), lambda i: (0, 0)),
        ],
        out_specs=pl.BlockSpec((_RB, dcols), lambda i: (i, 0)),
        out_shape=jax.ShapeDtypeStruct((N, dcols), jnp.float32),
    )(x, wf)


def _mm_relu(p0, p1, wf):
    din, dcols = wf.shape
    return pl.pallas_call(
        _mm_relu_body,
        grid=(N // _RB,),
        in_specs=[
            pl.BlockSpec((_RB, din), lambda i: (i, 0)),
            pl.BlockSpec((_RB, din), lambda i: (i, 0)),
            pl.BlockSpec((din, dcols), lambda i: (0, 0)),
        ],
        out_specs=pl.BlockSpec((_RB, dcols), lambda i: (i, 0)),
        out_shape=jax.ShapeDtypeStruct((N, dcols), jnp.float32),
    )(p0, p1, wf)


_BE = 2000


def _make_combine(DP, RW):
    def body(g_ref, w_ref, o_ref):
        g = g_ref[...]
        w = w_ref[0]
        acc = w[0][:, None] * g[:, 0 * DP:1 * DP]
        for j in range(1, 4):
            acc = acc + w[j][:, None] * g[:, j * DP:(j + 1) * DP]
        if DP < 128:
            acc = jnp.concatenate(
                [acc, jnp.zeros((_BE, 128 - DP), jnp.float32)], axis=1)
        o_ref[...] = acc

    def run(G, wblk):
        return pl.pallas_call(
            body,
            grid=(E // _BE,),
            in_specs=[
                pl.BlockSpec((_BE, RW), lambda i: (i, 0)),
                pl.BlockSpec((1, 4, _BE), lambda i: (i, 0, 0)),
            ],
            out_specs=pl.BlockSpec((_BE, 128), lambda i: (i, 0)),
            out_shape=jax.ShapeDtypeStruct((E, 128), jnp.float32),
        )(G, wblk)

    return run


def _fin_body(a_ref, b_ref, o_ref):
    o_ref[...] = (a_ref[...] + b_ref[...]) * SCALE


def _fin(p0r, p1r):
    return pl.pallas_call(
        _fin_body,
        out_shape=jax.ShapeDtypeStruct(p0r.shape, jnp.float32),
    )(p0r, p1r)


def _make_sc_gather(RW):
    @functools.partial(
        pl.kernel,
        out_type=jax.ShapeDtypeStruct((E, RW), jnp.float32),
        mesh=plsc.VectorSubcoreMesh(core_axis_name="c", subcore_axis_name="s"),
        scratch_types=[
            pltpu.VMEM((NB, C), jnp.int32),
            pltpu.VMEM((NB, C, RW), jnp.float32),
            pltpu.SemaphoreType.DMA,
        ],
    )
    def sc(p_hbm, idx_hbm, g_out, idx_v, g_v, gsem):
        cid = lax.axis_index("c")
        sid = lax.axis_index("s")
        wid = sid * NC + cid
        nch_w = (NCH // NW) + jnp.where(wid < NCH % NW, 1, 0)

        @pl.loop(0, nch_w, step=NB)
        def chunk(ch0):
            for b in range(NB):
                ch = ch0 + b
                eb = (wid + ch * NW) * C

                @pl.when(ch < nch_w)
                def _():
                    pltpu.sync_copy(idx_hbm.at[pl.ds(eb, C)], idx_v.at[b])
                    pltpu.async_copy(p_hbm.at[idx_v.at[b]], g_v.at[b],
                                     gsem).wait()
                    pltpu.sync_copy(g_v.at[b], g_out.at[pl.ds(eb, C)])

    return sc


def _make_sc_scatter():
    DV = 8

    @functools.partial(
        pl.kernel,
        out_type=jax.ShapeDtypeStruct((NC, N, 128), jnp.float32),
        mesh=plsc.VectorSubcoreMesh(core_axis_name="c", subcore_axis_name="s"),
        scratch_types=[
            pltpu.VMEM((C,), jnp.int32),
            pltpu.VMEM((C, 128), jnp.float32),
            pltpu.VMEM((ZR, 128), jnp.float32),
            pltpu.VMEM_SHARED((N, 128), jnp.float32),
        ],
    )
    def sc(msg_hbm, row_hbm, part_hbm, row_v, m_v, zbuf, acc_sh):
        cid = lax.axis_index("c")
        sid = lax.axis_index("s")
        wid = sid * NC + cid

        zero16 = jnp.zeros((16,), jnp.float32)

        @pl.loop(0, ZR)
        def zb(i):
            for v in range(DV):
                zbuf[i, pl.ds(v * 16, 16)] = zero16

        for i in range(RT // ZR):
            pltpu.sync_copy(zbuf, acc_sh.at[pl.ds(sid * RT + i * ZR, ZR)])

        @pl.when(sid == NS - 1)
        def _():
            pltpu.sync_copy(zbuf.at[pl.ds(0, 16)],
                            acc_sh.at[pl.ds(NS * RT, N - NS * RT)])

        plsc.subcore_barrier()
        nch_w = (NCH // NW) + jnp.where(wid < NCH % NW, 1, 0)

        @pl.loop(0, nch_w)
        def chunk(ch):
            eb = (wid + ch * NW) * C
            pltpu.sync_copy(msg_hbm.at[pl.ds(eb, C)], m_v)
            pltpu.sync_copy(row_hbm.at[pl.ds(eb, C)], row_v)
            pltpu.sync_copy(m_v, acc_sh.at[row_v], add=True)

        plsc.subcore_barrier()
        for i in range(RT // ZR):
            off = sid * RT + i * ZR
            pltpu.sync_copy(acc_sh.at[pl.ds(off, ZR)], zbuf)
            pltpu.sync_copy(zbuf, part_hbm.at[cid, pl.ds(off, ZR)])

        @pl.when(sid == NS - 1)
        def _():
            tail = N - NS * RT
            pltpu.sync_copy(acc_sh.at[pl.ds(NS * RT, tail)],
                            zbuf.at[pl.ds(0, tail)])
            pltpu.sync_copy(zbuf.at[pl.ds(0, tail)],
                            part_hbm.at[cid, pl.ds(NS * RT, tail)])

    return sc


_cache = {}


def _get(kind, *args):
    key = (kind,) + args
    if key not in _cache:
        maker = {"g": _make_sc_gather, "s": _make_sc_scatter,
                 "c": _make_combine}[kind]
        _cache[key] = maker(*args)
    return _cache[key]


_WOFF = (0, 1, 4, 5)


def _wf_dup(W, DP, RW):
    din = W.shape[1]
    if W.shape[2] < DP:
        W = jnp.pad(W, ((0, 0), (0, 0), (0, DP - W.shape[2])))
    kmap = jnp.array([[4 * cx + cy + o for o in _WOFF]
                      for cx in range(3) for cy in range(3)],
                     dtype=jnp.int32).reshape(-1)
    Wsel = W[kmap]
    Wsel = Wsel.transpose(1, 0, 2).reshape(din, 9, 4 * DP)
    if RW > 4 * DP:
        Wsel = jnp.pad(Wsel, ((0, 0), (0, 0), (0, RW - 4 * DP)))
    return Wsel.reshape(din, 9 * RW)


def _layer(table, idx, wblk, row, DP, RW):
    G = _get("g", RW)(table, idx)
    msg = _get("c", DP, RW)(G, wblk)
    return _get("s")(msg, row)


def kernel(fluidFeatures, fluid_edge_index, fluid_edge_distances,
           W0, W1, W2, W3):
    row = fluid_edge_index[0]
    col = fluid_edge_index[1]
    col_r = col.reshape(_ER, _EC)
    dx_r = fluid_edge_distances[:, 0].reshape(_ER, _EC)
    dy_r = fluid_edge_distances[:, 1].reshape(_ER, _EC)
    idx1, w4 = _edge_prep(col_r, dx_r, dy_r)
    idx = idx1.reshape(E)
    wblk = w4.reshape(4, E // _BE, _BE).transpose(1, 0, 2)

    table = _mm0(fluidFeatures, _wf_dup(W0, 32, 128)).reshape(9 * N, 128)
    part = _layer(table, idx, wblk, row, 32, 128)

    table = _mm_relu(part[0][:, :32], part[1][:, :32],
                     _wf_dup(W1, 64, 256)).reshape(9 * N, 256)
    part = _layer(table, idx, wblk, row, 64, 256)

    table = _mm_relu(part[0][:, :64], part[1][:, :64],
                     _wf_dup(W2, 64, 256)).reshape(9 * N, 256)
    part = _layer(table, idx, wblk, row, 64, 256)

    table = _mm_relu(part[0][:, :64], part[1][:, :64],
                     _wf_dup(W3, 16, 128)).reshape(9 * N, 128)
    part = _layer(table, idx, wblk, row, 16, 128)

    y = _fin(part[0], part[1])
    return y[:, :2]


# speedup vs baseline: 2.8708x; 1.2074x over previous
import functools

import jax
import jax.numpy as jnp
from jax import lax
from jax.experimental import pallas as pl
from jax.experimental.pallas import tpu as pltpu
from jax.experimental.pallas import tpu_sc as plsc

N = 10000
E = 320000
SCALE = 1.0 / 128.0

NC = 2
NS = 16
NW = NC * NS
C = 80
NCH = E // C
RT = 624
ZR = 208
NB = 2


def _prep_body(col_ref, dx_ref, dy_ref, idx_ref, w_ref):
    col = col_ref[...]
    dx = dx_ref[...]
    dy = dy_ref[...]
    tx = (dx + 1.0) * 1.5
    ty = (dy + 1.0) * 1.5
    cxf = jnp.clip(jnp.floor(tx), 0.0, 2.0)
    cyf = jnp.clip(jnp.floor(ty), 0.0, 2.0)
    fx = tx - cxf
    fy = ty - cyf
    cx = cxf.astype(jnp.int32)
    cy = cyf.astype(jnp.int32)
    idx_ref[0] = col * 9 + cx * 3 + cy
    wx0 = 1.0 - fx
    wy0 = 1.0 - fy
    w_ref[0] = wx0 * wy0
    w_ref[1] = wx0 * fy
    w_ref[2] = fx * wy0
    w_ref[3] = fx * fy


_ER, _EC = 2500, 128


def _edge_prep(col_r, dx_r, dy_r):
    return pl.pallas_call(
        _prep_body,
        out_shape=(
            jax.ShapeDtypeStruct((1, _ER, _EC), jnp.int32),
            jax.ShapeDtypeStruct((4, _ER, _EC), jnp.float32),
        ),
    )(col_r, dx_r, dy_r)


def _mm_body(x_ref, w_ref, o_ref):
    o_ref[...] = jnp.dot(x_ref[...], w_ref[...],
                         preferred_element_type=jnp.float32)


def _mm_relu_body(a_ref, b_ref, w_ref, o_ref):
    x = jnp.maximum(a_ref[...] + b_ref[...], 0.0)
    o_ref[...] = jnp.dot(x, w_ref[...], preferred_element_type=jnp.float32)


_RB = 1000


def _mm0(x, wf):
    din, dcols = wf.shape
    return pl.pallas_call(
        _mm_body,
        grid=(N // _RB,),
        in_specs=[
            pl.BlockSpec((_RB, din), lambda i: (i, 0)),
            pl.BlockSpec((din, dcols), lambda i: (0, 0)),
        ],
        out_specs=pl.BlockSpec((_RB, dcols), lambda i: (i, 0)),
        out_shape=jax.ShapeDtypeStruct((N, dcols), jnp.float32),
    )(x, wf)


def _mm_relu(p0, p1, wf):
    din, dcols = wf.shape
    return pl.pallas_call(
        _mm_relu_body,
        grid=(N // _RB,),
        in_specs=[
            pl.BlockSpec((_RB, din), lambda i: (i, 0)),
            pl.BlockSpec((_RB, din), lambda i: (i, 0)),
            pl.BlockSpec((din, dcols), lambda i: (0, 0)),
        ],
        out_specs=pl.BlockSpec((_RB, dcols), lambda i: (i, 0)),
        out_shape=jax.ShapeDtypeStruct((N, dcols), jnp.float32),
    )(p0, p1, wf)


_BE = 2000


def _make_combine(DP, RW):
    def body(g_ref, w_ref, o_ref):
        g = g_ref[...]
        w = w_ref[0]
        acc = w[0][:, None] * g[:, 0 * DP:1 * DP]
        for j in range(1, 4):
            acc = acc + w[j][:, None] * g[:, j * DP:(j + 1) * DP]
        if DP < 128:
            acc = jnp.concatenate(
                [acc, jnp.zeros((_BE, 128 - DP), jnp.float32)], axis=1)
        o_ref[...] = acc

    def run(G, wblk):
        return pl.pallas_call(
            body,
            grid=(E // _BE,),
            in_specs=[
                pl.BlockSpec((_BE, RW), lambda i: (i, 0)),
                pl.BlockSpec((1, 4, _BE), lambda i: (i, 0, 0)),
            ],
            out_specs=pl.BlockSpec((_BE, 128), lambda i: (i, 0)),
            out_shape=jax.ShapeDtypeStruct((E, 128), jnp.float32),
        )(G, wblk)

    return run


def _fin_body(a_ref, b_ref, o_ref):
    o_ref[...] = (a_ref[...] + b_ref[...]) * SCALE


def _fin(p0r, p1r):
    return pl.pallas_call(
        _fin_body,
        out_shape=jax.ShapeDtypeStruct(p0r.shape, jnp.float32),
    )(p0r, p1r)


def _make_sc_gather(RW):
    @functools.partial(
        pl.kernel,
        out_type=jax.ShapeDtypeStruct((E, RW), jnp.float32),
        mesh=plsc.VectorSubcoreMesh(core_axis_name="c", subcore_axis_name="s"),
        scratch_types=[
            pltpu.VMEM((NB, C), jnp.int32),
            pltpu.VMEM((NB, C, RW), jnp.float32),
            pltpu.SemaphoreType.DMA,
            pltpu.SemaphoreType.DMA,
        ],
    )
    def sc(p_hbm, idx_hbm, g_out, idx_v, g_v, gsem, wsem):
        cid = lax.axis_index("c")
        sid = lax.axis_index("s")
        wid = sid * NC + cid
        nch_w = (NCH // NW) + jnp.where(wid < NCH % NW, 1, 0)

        @pl.loop(0, nch_w, step=NB)
        def chunk(ch0):
            for b in range(NB):
                ch = ch0 + b
                eb = (wid + ch * NW) * C

                @pl.when(ch < nch_w)
                def _():
                    @pl.when(ch >= NB)
                    def _():
                        pltpu.make_async_copy(
                            g_v.at[b], g_out.at[pl.ds(eb, C)], wsem).wait()

                    pltpu.sync_copy(idx_hbm.at[pl.ds(eb, C)], idx_v.at[b])
                    pltpu.async_copy(p_hbm.at[idx_v.at[b]], g_v.at[b],
                                     gsem).wait()
                    pltpu.async_copy(g_v.at[b], g_out.at[pl.ds(eb, C)],
                                     wsem)

        for b in range(NB):
            eb = (wid + b * NW) * C

            @pl.when(b < nch_w)
            def _():
                pltpu.make_async_copy(
                    g_v.at[b], g_out.at[pl.ds(eb, C)], wsem).wait()

    return sc


def _make_sc_scatter():
    DV = 8

    @functools.partial(
        pl.kernel,
        out_type=jax.ShapeDtypeStruct((NC, N, 128), jnp.float32),
        mesh=plsc.VectorSubcoreMesh(core_axis_name="c", subcore_axis_name="s"),
        scratch_types=[
            pltpu.VMEM((C,), jnp.int32),
            pltpu.VMEM((C,), jnp.int32),
            pltpu.VMEM((NB, C, 128), jnp.float32),
            pltpu.VMEM((ZR, 128), jnp.float32),
            pltpu.VMEM_SHARED((N, 128), jnp.float32),
            pltpu.SemaphoreType.DMA,
        ],
    )
    def sc(msg_hbm, row_hbm, part_hbm, row_va, row_vb, m_v, zbuf, acc_sh,
           ssem):
        cid = lax.axis_index("c")
        sid = lax.axis_index("s")
        wid = sid * NC + cid

        zero16 = jnp.zeros((16,), jnp.float32)

        @pl.loop(0, ZR)
        def zb(i):
            for v in range(DV):
                zbuf[i, pl.ds(v * 16, 16)] = zero16

        for i in range(RT // ZR):
            pltpu.sync_copy(zbuf, acc_sh.at[pl.ds(sid * RT + i * ZR, ZR)])

        @pl.when(sid == NS - 1)
        def _():
            pltpu.sync_copy(zbuf.at[pl.ds(0, 16)],
                            acc_sh.at[pl.ds(NS * RT, N - NS * RT)])

        plsc.subcore_barrier()
        nch_w = (NCH // NW) + jnp.where(wid < NCH % NW, 1, 0)

        rv = (row_va, row_vb)

        @pl.loop(0, nch_w, step=NB)
        def chunk(ch0):
            for b in range(NB):
                ch = ch0 + b
                eb = (wid + ch * NW) * C

                @pl.when(ch < nch_w)
                def _():
                    @pl.when(ch >= NB)
                    def _():
                        pltpu.make_async_copy(
                            msg_hbm.at[pl.ds(eb, C)], m_v.at[b],
                            ssem).wait()

                    pltpu.sync_copy(msg_hbm.at[pl.ds(eb, C)], m_v.at[b])
                    pltpu.sync_copy(row_hbm.at[pl.ds(eb, C)], rv[b])
                    pltpu.async_copy(m_v.at[b], acc_sh.at[rv[b]], ssem,
                                     add=True)

        for b in range(NB):
            eb = (wid + b * NW) * C

            @pl.when(b < nch_w)
            def _():
                pltpu.make_async_copy(msg_hbm.at[pl.ds(eb, C)], m_v.at[b],
                                      ssem).wait()

        plsc.subcore_barrier()
        for i in range(RT // ZR):
            off = sid * RT + i * ZR
            pltpu.sync_copy(acc_sh.at[pl.ds(off, ZR)], zbuf)
            pltpu.sync_copy(zbuf, part_hbm.at[cid, pl.ds(off, ZR)])

        @pl.when(sid == NS - 1)
        def _():
            tail = N - NS * RT
            pltpu.sync_copy(acc_sh.at[pl.ds(NS * RT, tail)],
                            zbuf.at[pl.ds(0, tail)])
            pltpu.sync_copy(zbuf.at[pl.ds(0, tail)],
                            part_hbm.at[cid, pl.ds(NS * RT, tail)])

    return sc


_cache = {}


def _get(kind, *args):
    key = (kind,) + args
    if key not in _cache:
        maker = {"g": _make_sc_gather, "s": _make_sc_scatter,
                 "c": _make_combine}[kind]
        _cache[key] = maker(*args)
    return _cache[key]


_WOFF = (0, 1, 4, 5)


def _wf_dup(W, DP, RW):
    din = W.shape[1]
    if W.shape[2] < DP:
        W = jnp.pad(W, ((0, 0), (0, 0), (0, DP - W.shape[2])))
    kmap = jnp.array([[4 * cx + cy + o for o in _WOFF]
                      for cx in range(3) for cy in range(3)],
                     dtype=jnp.int32).reshape(-1)
    Wsel = W[kmap]
    Wsel = Wsel.transpose(1, 0, 2).reshape(din, 9, 4 * DP)
    if RW > 4 * DP:
        Wsel = jnp.pad(Wsel, ((0, 0), (0, 0), (0, RW - 4 * DP)))
    return Wsel.reshape(din, 9 * RW)


def _layer(table, idx, wblk, row, DP, RW):
    G = _get("g", RW)(table, idx)
    msg = _get("c", DP, RW)(G, wblk)
    return _get("s")(msg, row)


def kernel(fluidFeatures, fluid_edge_index, fluid_edge_distances,
           W0, W1, W2, W3):
    row = fluid_edge_index[0]
    col = fluid_edge_index[1]
    col_r = col.reshape(_ER, _EC)
    dx_r = fluid_edge_distances[:, 0].reshape(_ER, _EC)
    dy_r = fluid_edge_distances[:, 1].reshape(_ER, _EC)
    idx1, w4 = _edge_prep(col_r, dx_r, dy_r)
    idx = idx1.reshape(E)
    wblk = w4.reshape(4, E // _BE, _BE).transpose(1, 0, 2)

    table = _mm0(fluidFeatures, _wf_dup(W0, 32, 128)).reshape(9 * N, 128)
    part = _layer(table, idx, wblk, row, 32, 128)

    table = _mm_relu(part[0][:, :32], part[1][:, :32],
                     _wf_dup(W1, 64, 256)).reshape(9 * N, 256)
    part = _layer(table, idx, wblk, row, 64, 256)

    table = _mm_relu(part[0][:, :64], part[1][:, :64],
                     _wf_dup(W2, 64, 256)).reshape(9 * N, 256)
    part = _layer(table, idx, wblk, row, 64, 256)

    table = _mm_relu(part[0][:, :64], part[1][:, :64],
                     _wf_dup(W3, 16, 128)).reshape(9 * N, 128)
    part = _layer(table, idx, wblk, row, 16, 128)

    y = _fin(part[0], part[1])
    return y[:, :2]
